# Initial kernel scaffold; baseline (speedup 1.0000x reference)
import jax
import jax.numpy as jnp
from jax.experimental import pallas as pl


def kernel(Z, Y):
    raise NotImplementedError("write your pallas kernel here")


# speedup vs baseline: 8.4499x; 8.4499x over previous
import functools

import jax
import jax.numpy as jnp
from jax.experimental import pallas as pl
from jax.experimental.pallas import tpu as pltpu


def _acc_kernel(z_ref, y_ref, out_ref):
    x = z_ref[:]
    y = y_ref[:]
    n = x.shape[0]
    num = jax.lax.dot_general(
        x, y,
        dimension_numbers=(((1,), (1,)), ((), ())),
        preferred_element_type=jnp.float32,
    )
    xn = jnp.sqrt(jnp.sum(x * x, axis=1))
    yn = jnp.sqrt(jnp.sum(y * y, axis=1))
    denom = jnp.maximum(xn[:, None] * yn[None, :], 1e-8)
    simt = num / denom
    row = jax.lax.broadcasted_iota(jnp.int32, (n, n), 0)
    col = jax.lax.broadcasted_iota(jnp.int32, (n, n), 1)
    diag_mask = row == col
    d = jnp.sum(jnp.where(diag_mask, simt, 0.0), axis=0, keepdims=True)
    beats = (simt > d) | ((simt == d) & (row < col))
    cnt = jnp.sum(beats.astype(jnp.float32), axis=0, keepdims=True)
    top1 = jnp.sum((cnt == 0.0).astype(jnp.float32), axis=1, keepdims=True)
    top10 = jnp.sum((cnt < 10.0).astype(jnp.float32), axis=1, keepdims=True)
    out_ref[...] = jnp.concatenate([top1, top10], axis=1) * (1.0 / n)


@functools.partial(jax.jit, static_argnames=())
def kernel(Z, Y):
    out = pl.pallas_call(
        _acc_kernel,
        out_shape=jax.ShapeDtypeStruct((1, 2), jnp.float32),
    )(Z, Y)
    return (out[0, 0], out[0, 1])
